# Initial kernel scaffold; baseline (speedup 1.0000x reference)
import jax
import jax.numpy as jnp
from jax.experimental import pallas as pl


def kernel(x, edge_index, batch, W1, b1, W2, b2, Wl, bl):
    raise NotImplementedError("write your pallas kernel here")


# speedup vs baseline: 16.3257x; 16.3257x over previous
import functools

import jax
import jax.numpy as jnp
from jax import lax
from jax.experimental import pallas as pl
from jax.experimental.pallas import tpu as pltpu
from jax.experimental.pallas import tpu_sc as plsc

N, E, D, G, C = 10000, 320000, 128, 64, 2
NC, NS = 2, 16
NW = NC * NS
EPW = E // NW
CHUNK = 100
NCHUNK = EPW // CHUNK
NPAD = 10240
RPS = NPAD // NS
ZROWS = 64
DEGW = 128

_MESH = dict(core_axis_name="c", subcore_axis_name="s", num_cores=NC,
             num_subcores=NS)


def _zero_fill(buf, rows, cols):
    @pl.loop(0, rows)
    def _(i):
        @pl.loop(0, cols, step=16)
        def _(j):
            buf[i, pl.ds(j, 16)] = jnp.zeros((16,), jnp.float32)


def _sc_deg(row3, interpret=False):
    mesh = plsc.VectorSubcoreMesh(**_MESH)

    @functools.partial(
        pl.kernel,
        out_type=jax.ShapeDtypeStruct((NC, NPAD, DEGW), jnp.float32),
        mesh=mesh,
        interpret=interpret,
        scratch_types=[
            pltpu.VMEM_SHARED((NPAD, DEGW), jnp.float32),
            pltpu.VMEM((NCHUNK, CHUNK), jnp.int32),
            pltpu.VMEM((CHUNK, DEGW), jnp.float32),
            pltpu.VMEM((ZROWS, DEGW), jnp.float32),
        ],
    )
    def k(row_hbm, out_hbm, acc, row_v, ones_v, zbuf):
        c = lax.axis_index("c")
        s = lax.axis_index("s")
        wid = s * NC + c
        _zero_fill(zbuf, ZROWS, DEGW)
        @pl.loop(0, RPS, step=ZROWS)
        def _(r):
            pltpu.sync_copy(zbuf, acc.at[pl.ds(s * RPS + r, ZROWS)])
        @pl.loop(0, CHUNK)
        def _(i):
            @pl.loop(0, DEGW, step=16)
            def _(j):
                ones_v[i, pl.ds(j, 16)] = jnp.ones((16,), jnp.float32)
        pltpu.sync_copy(row_hbm.at[wid], row_v)
        plsc.subcore_barrier()
        @pl.loop(0, NCHUNK)
        def _(j):
            pltpu.sync_copy(ones_v, acc.at[row_v.at[j]], add=True)
        plsc.subcore_barrier()
        pltpu.sync_copy(acc.at[pl.ds(s * RPS, RPS)],
                        out_hbm.at[c].at[pl.ds(s * RPS, RPS)])

    return k(row3)


def _sc_agg(u, row3, col3, interpret=False):
    mesh = plsc.VectorSubcoreMesh(**_MESH)

    @functools.partial(
        pl.kernel,
        out_type=jax.ShapeDtypeStruct((NC, NPAD, D), jnp.float32),
        mesh=mesh,
        interpret=interpret,
        scratch_types=[
            pltpu.VMEM_SHARED((NPAD, D), jnp.float32),
            pltpu.VMEM((NCHUNK, CHUNK), jnp.int32),
            pltpu.VMEM((NCHUNK, CHUNK), jnp.int32),
            pltpu.VMEM((CHUNK, D), jnp.float32),
            pltpu.VMEM((ZROWS, D), jnp.float32),
        ],
    )
    def k(u_hbm, row_hbm, col_hbm, out_hbm, acc, row_v, col_v, gbuf, zbuf):
        c = lax.axis_index("c")
        s = lax.axis_index("s")
        wid = s * NC + c
        _zero_fill(zbuf, ZROWS, D)
        @pl.loop(0, RPS, step=ZROWS)
        def _(r):
            pltpu.sync_copy(zbuf, acc.at[pl.ds(s * RPS + r, ZROWS)])
        pltpu.sync_copy(row_hbm.at[wid], row_v)
        pltpu.sync_copy(col_hbm.at[wid], col_v)
        plsc.subcore_barrier()
        @pl.loop(0, NCHUNK)
        def _(j):
            pltpu.sync_copy(u_hbm.at[col_v.at[j]], gbuf)
            pltpu.sync_copy(gbuf, acc.at[row_v.at[j]], add=True)
        plsc.subcore_barrier()
        pltpu.sync_copy(acc.at[pl.ds(s * RPS, RPS)],
                        out_hbm.at[c].at[pl.ds(s * RPS, RPS)])

    return k(u, row3, col3)


BN = 1000


def _d_of(dp_ref):
    deg = dp_ref[0, :, 0] + dp_ref[1, :, 0]
    return jnp.where(deg > 0, lax.rsqrt(deg), 0.0)


def _tc_mm(x, wt, interpret=False):
    def body(x_ref, w_ref, o_ref):
        o_ref[...] = jnp.dot(x_ref[...], w_ref[...],
                             preferred_element_type=jnp.float32)

    return pl.pallas_call(
        body,
        grid=(N // BN,),
        in_specs=[
            pl.BlockSpec((BN, D), lambda i: (i, 0)),
            pl.BlockSpec((D, D), lambda i: (0, 0)),
        ],
        out_specs=pl.BlockSpec((BN, D), lambda i: (i, 0)),
        out_shape=jax.ShapeDtypeStruct((N, D), jnp.float32),
        interpret=interpret,
    )(x, wt)


def _tc_scale(y, degp, interpret=False):
    def body(y_ref, dp_ref, o_ref):
        d = _d_of(dp_ref)
        o_ref[...] = y_ref[...] * d[:, None]

    return pl.pallas_call(
        body,
        grid=(N // BN,),
        in_specs=[
            pl.BlockSpec((BN, D), lambda i: (i, 0)),
            pl.BlockSpec((NC, BN, DEGW), lambda i: (0, i, 0)),
        ],
        out_specs=pl.BlockSpec((BN, D), lambda i: (i, 0)),
        out_shape=jax.ShapeDtypeStruct((N, D), jnp.float32),
        interpret=interpret,
    )(y, degp)


def _tc_dense2(parts, degp, b1, w2t, interpret=False):
    def body(p_ref, dp_ref, b_ref, w_ref, o_ref):
        d = _d_of(dp_ref)
        agg = p_ref[0] + p_ref[1]
        h = jnp.maximum(agg * d[:, None] + b_ref[...], 0.0)
        o_ref[...] = jnp.dot(h, w_ref[...],
                             preferred_element_type=jnp.float32) * d[:, None]

    return pl.pallas_call(
        body,
        grid=(N // BN,),
        in_specs=[
            pl.BlockSpec((NC, BN, D), lambda i: (0, i, 0)),
            pl.BlockSpec((NC, BN, DEGW), lambda i: (0, i, 0)),
            pl.BlockSpec((1, D), lambda i: (0, 0)),
            pl.BlockSpec((D, D), lambda i: (0, 0)),
        ],
        out_specs=pl.BlockSpec((BN, D), lambda i: (i, 0)),
        out_shape=jax.ShapeDtypeStruct((N, D), jnp.float32),
        interpret=interpret,
    )(parts, degp, b1.reshape(1, D), w2t)


def _tc_final(parts, degp, b2, batch3, wlt, bl, interpret=False):
    def body(p_ref, dp_ref, b_ref, seg_ref, wl_ref, bl_ref, o_ref, pool_ref):
        i = pl.program_id(0)

        @pl.when(i == 0)
        def _():
            pool_ref[...] = jnp.zeros_like(pool_ref)

        d = _d_of(dp_ref)
        agg = p_ref[0] + p_ref[1]
        h = jnp.maximum(agg * d[:, None] + b_ref[...], 0.0)
        seg = seg_ref[0, 0, :]
        onehot = (seg[None, :] ==
                  lax.broadcasted_iota(jnp.int32, (G, BN), 0)).astype(jnp.float32)
        pool_ref[...] += jnp.dot(onehot, h, preferred_element_type=jnp.float32)

        @pl.when(i == pl.num_programs(0) - 1)
        def _():
            o_ref[...] = (jnp.dot(pool_ref[...], wl_ref[...],
                                  preferred_element_type=jnp.float32)
                          + bl_ref[...])

    return pl.pallas_call(
        body,
        grid=(N // BN,),
        in_specs=[
            pl.BlockSpec((NC, BN, D), lambda i: (0, i, 0)),
            pl.BlockSpec((NC, BN, DEGW), lambda i: (0, i, 0)),
            pl.BlockSpec((1, D), lambda i: (0, 0)),
            pl.BlockSpec((1, 1, BN), lambda i: (i, 0, 0)),
            pl.BlockSpec((D, C), lambda i: (0, 0)),
            pl.BlockSpec((1, C), lambda i: (0, 0)),
        ],
        out_specs=pl.BlockSpec((G, C), lambda i: (0, 0)),
        out_shape=jax.ShapeDtypeStruct((G, C), jnp.float32),
        scratch_shapes=[pltpu.VMEM((G, D), jnp.float32)],
        interpret=interpret,
    )(parts, degp, b2.reshape(1, D), batch3, wlt, bl.reshape(1, C))


def _gnn(x, edge_index, batch, W1, b1, W2, b2, Wl, bl, interpret=False):
    row3 = edge_index[0].reshape(NW, NCHUNK, CHUNK)
    col3 = edge_index[1].reshape(NW, NCHUNK, CHUNK)
    batch3 = batch.reshape(N // BN, 1, BN)

    degp = _sc_deg(row3, interpret=interpret)
    y1 = _tc_mm(x, W1.T, interpret=interpret)
    u1 = _tc_scale(y1, degp, interpret=interpret)
    parts1 = _sc_agg(u1, row3, col3, interpret=interpret)
    u2 = _tc_dense2(parts1, degp, b1, W2.T, interpret=interpret)
    parts2 = _sc_agg(u2, row3, col3, interpret=interpret)
    return _tc_final(parts2, degp, b2, batch3, Wl.T, bl, interpret=interpret)


def kernel(x, edge_index, batch, W1, b1, W2, b2, Wl, bl):
    return _gnn(x, edge_index, batch, W1, b1, W2, b2, Wl, bl)
